# Initial kernel scaffold; baseline (speedup 1.0000x reference)
import jax
import jax.numpy as jnp
from jax.experimental import pallas as pl


def kernel(x1, edge_index1, edge_attr1, x2, edge_index2, edge_attr2, W1, b1, W2, b2, Wc1, bc1, Wc2, bc2, Wc3, bc3):
    raise NotImplementedError("write your pallas kernel here")


# speedup vs baseline: 32.1577x; 32.1577x over previous
import functools

import jax
import jax.numpy as jnp
from jax import lax
from jax.experimental import pallas as pl
from jax.experimental.pallas import tpu as pltpu
from jax.experimental.pallas import tpu_sc as plsc

_N = 200
_E = 20000
_NP = 256
_NT = 32
_RPG = 5
_ROWS = 2 * _RPG
_EPT = 128 * _RPG
_EPAD = _NT * _EPT
_ASZ = _NP * _NP
_ACC = 2 * _ASZ
_SLC = _ACC // 16


def _sc_build_adj(src, dst, ew, zeros_slice):
    mesh = plsc.VectorSubcoreMesh(core_axis_name="c", subcore_axis_name="s")

    @functools.partial(
        pl.kernel,
        mesh=mesh,
        out_type=jax.ShapeDtypeStruct((2, _ACC), jnp.float32),
        scratch_types=[
            pltpu.VMEM((_ROWS, 128), jnp.int32),
            pltpu.VMEM((_ROWS, 128), jnp.int32),
            pltpu.VMEM((_ROWS, 128), jnp.float32),
            pltpu.VMEM((_ROWS, 128), jnp.int32),
            pltpu.VMEM_SHARED((_ACC,), jnp.float32),
        ],
    )
    def k(src_hbm, dst_hbm, ew_hbm, zero_hbm, out_hbm, src_v, dst_v, ew_v,
          idx_v, acc):
        cid = lax.axis_index("c")
        sid = lax.axis_index("s")
        wid = sid * 2 + cid
        pltpu.sync_copy(zero_hbm, acc.at[pl.ds(sid * _SLC, _SLC)])
        pltpu.sync_copy(src_hbm.at[wid], src_v)
        pltpu.sync_copy(dst_hbm.at[wid], dst_v)
        pltpu.sync_copy(ew_hbm.at[wid], ew_v)
        for c in range(_ROWS):
            goff = 0 if c < _RPG else _ASZ
            for j in range(8):
                s16 = src_v[c, pl.ds(j * 16, 16)]
                d16 = dst_v[c, pl.ds(j * 16, 16)]
                idx_v[c, pl.ds(j * 16, 16)] = d16 * _NP + s16 + goff
        plsc.subcore_barrier()
        for c in range(_ROWS):
            pltpu.sync_copy(ew_v.at[c], acc.at[idx_v.at[c]], add=True)
        plsc.subcore_barrier()
        pltpu.sync_copy(acc.at[pl.ds(sid * _SLC, _SLC)],
                        out_hbm.at[cid, pl.ds(sid * _SLC, _SLC)])

    return k(src, dst, ew, zeros_slice)


def _tc_forward(parts, x1, x2, W1, b1, W2, b2, Wc1p, bc1, Wc2, bc2, Wc3,
                bc3):
    def body(parts_ref, x1_ref, x2_ref, W1_ref, b1_ref, W2_ref, b2_ref,
             Wc1_ref, bc1_ref, Wc2_ref, bc2_ref, Wc3_ref, bc3_ref,
             out1_ref, out2_ref):
        hi = lax.Precision.HIGHEST

        def mm(a, b):
            return lax.dot_general(a, b, (((1,), (0,)), ((), ())),
                                   precision=hi,
                                   preferred_element_type=jnp.float32)

        b1v = b1_ref[...]
        b2v = b2_ref[...]
        Wc1 = Wc1_ref[...]
        Wc2 = Wc2_ref[...]
        Wc3 = Wc3_ref[...]
        bc1 = bc1_ref[...]
        bc2 = bc2_ref[...]
        bc3 = bc3_ref[...]

        def run(A, x):
            deg = jnp.sum(A, axis=0)
            dinv = jnp.where(deg > 0.0, lax.rsqrt(deg), 0.0)
            S = -(dinv[:, None] * A * dinv[None, :])
            T1 = mm(S, x)
            T2 = 2.0 * mm(S, T1) - x
            h = (mm(x, W1_ref[0]) + mm(T1, W1_ref[1]) + mm(T2, W1_ref[2])
                 + b1v)
            h = jnp.maximum(h, 0.0)
            U1 = mm(S, h)
            U2 = 2.0 * mm(S, U1) - h
            o = (mm(h, W2_ref[0]) + mm(U1, W2_ref[1]) + mm(U2, W2_ref[2])
                 + b2v)
            c1 = lax.dot_general(o, Wc1, (((0,), (0,)), ((), ())),
                                 precision=hi,
                                 preferred_element_type=jnp.float32)
            c1 = jnp.maximum(c1 + bc1, 0.0)
            c2 = jnp.maximum(mm(c1, Wc2) + bc2, 0.0)
            return mm(c2, Wc3) + bc3

        A1 = parts_ref[0, 0] + parts_ref[1, 0]
        A2 = parts_ref[0, 1] + parts_ref[1, 1]
        out1_ref[...] = run(A1, x1_ref[...])
        out2_ref[...] = run(A2, x2_ref[...])

    return pl.pallas_call(
        body,
        out_shape=[jax.ShapeDtypeStruct((2, 1), jnp.float32)] * 2,
    )(parts, x1, x2, W1, b1, W2, b2, Wc1p, bc1, Wc2, bc2, Wc3, bc3)


def kernel(x1, edge_index1, edge_attr1, x2, edge_index2, edge_attr2, W1, b1,
           W2, b2, Wc1, bc1, Wc2, bc2, Wc3, bc3):
    def prep(ei, ew):
        srcp = jnp.pad(ei[0].astype(jnp.int32), (0, _EPAD - _E))
        dstp = jnp.pad(ei[1].astype(jnp.int32), (0, _EPAD - _E))
        ewp = jnp.pad(ew.astype(jnp.float32), (0, _EPAD - _E))
        return (srcp.reshape(_NT, _RPG, 128), dstp.reshape(_NT, _RPG, 128),
                ewp.reshape(_NT, _RPG, 128))

    s1, d1, e1 = prep(edge_index1, edge_attr1)
    s2, d2, e2 = prep(edge_index2, edge_attr2)
    src = jnp.concatenate([s1, s2], axis=1)
    dst = jnp.concatenate([d1, d2], axis=1)
    ewc = jnp.concatenate([e1, e2], axis=1)
    zeros = jnp.zeros((_SLC,), jnp.float32)
    parts = _sc_build_adj(src, dst, ewc, zeros).reshape(2, 2, _NP, _NP)
    xp1 = jnp.pad(x1, ((0, _NP - _N), (0, 0)))
    xp2 = jnp.pad(x2, ((0, _NP - _N), (0, 0)))
    Wc1p = jnp.pad(Wc1, ((0, _NP - _N), (0, 0)))
    out1, out2 = _tc_forward(parts, xp1, xp2, W1, b1, W2, b2, Wc1p, bc1,
                             Wc2, bc2, Wc3, bc3)
    return (out1, out2)


# speedup vs baseline: 43.2591x; 1.3452x over previous
import functools

import jax
import jax.numpy as jnp
from jax import lax
from jax.experimental import pallas as pl
from jax.experimental.pallas import tpu as pltpu
from jax.experimental.pallas import tpu_sc as plsc

_N = 200
_E = 20000
_NP = 256
_NT = 32
_RPG = 5
_ROWS = 2 * _RPG
_EPT = 128 * _RPG
_EPAD = _NT * _EPT
_ASZ = _NP * _NP
_ACC = 2 * _ASZ
_SLC = _ACC // 16


def _sc_build_adj(src, dst, ew, zeros_slice):
    mesh = plsc.VectorSubcoreMesh(core_axis_name="c", subcore_axis_name="s")

    @functools.partial(
        pl.kernel,
        mesh=mesh,
        out_type=jax.ShapeDtypeStruct((2, _ACC), jnp.float32),
        scratch_types=[
            pltpu.VMEM((_ROWS, 128), jnp.int32),
            pltpu.VMEM((_ROWS, 128), jnp.int32),
            pltpu.VMEM((_ROWS, 128), jnp.float32),
            pltpu.VMEM((_ROWS, 128), jnp.int32),
            pltpu.VMEM_SHARED((_ACC,), jnp.float32),
            pltpu.SemaphoreType.DMA,
        ],
    )
    def k(src_hbm, dst_hbm, ew_hbm, zero_hbm, out_hbm, src_v, dst_v, ew_v,
          idx_v, acc, sem):
        cid = lax.axis_index("c")
        sid = lax.axis_index("s")
        wid = sid * 2 + cid
        stage = [
            pltpu.async_copy(zero_hbm, acc.at[pl.ds(sid * _SLC, _SLC)], sem),
            pltpu.async_copy(src_hbm.at[wid], src_v, sem),
            pltpu.async_copy(dst_hbm.at[wid], dst_v, sem),
            pltpu.async_copy(ew_hbm.at[wid], ew_v, sem),
        ]
        for cp in stage:
            cp.wait()
        for c in range(_ROWS):
            goff = 0 if c < _RPG else _ASZ
            for j in range(8):
                s16 = src_v[c, pl.ds(j * 16, 16)]
                d16 = dst_v[c, pl.ds(j * 16, 16)]
                idx_v[c, pl.ds(j * 16, 16)] = d16 * _NP + s16 + goff
        plsc.subcore_barrier()
        adds = [pltpu.async_copy(ew_v.at[c], acc.at[idx_v.at[c]], sem,
                                 add=True) for c in range(_ROWS)]
        for cp in adds:
            cp.wait()
        plsc.subcore_barrier()
        pltpu.sync_copy(acc.at[pl.ds(sid * _SLC, _SLC)],
                        out_hbm.at[cid, pl.ds(sid * _SLC, _SLC)])

    return k(src, dst, ew, zeros_slice)


def _tc_forward(parts, x1, x2, W1, b1, W2, b2, Wc1p, bc1, Wc2, bc2, Wc3,
                bc3):
    def body(parts_ref, x1_ref, x2_ref, W1_ref, b1_ref, W2_ref, b2_ref,
             Wc1_ref, bc1_ref, Wc2_ref, bc2_ref, Wc3_ref, bc3_ref,
             out1_ref, out2_ref):
        hi = lax.Precision.DEFAULT

        def mm(a, b):
            return lax.dot_general(a, b, (((1,), (0,)), ((), ())),
                                   precision=hi,
                                   preferred_element_type=jnp.float32)

        b1v = b1_ref[...]
        b2v = b2_ref[...]
        Wc1 = jnp.pad(Wc1_ref[...], ((0, _NP - _N), (0, 0)))
        Wc2 = Wc2_ref[...]
        Wc3 = Wc3_ref[...]
        bc1 = bc1_ref[...]
        bc2 = bc2_ref[...]
        bc3 = bc3_ref[...]

        def run(A, x):
            deg = jnp.sum(A, axis=0)
            dinv = jnp.where(deg > 0.0, lax.rsqrt(deg), 0.0)
            S = -(dinv[:, None] * A * dinv[None, :])
            T1 = mm(S, x)
            T2 = 2.0 * mm(S, T1) - x
            h = (mm(x, W1_ref[0]) + mm(T1, W1_ref[1]) + mm(T2, W1_ref[2])
                 + b1v)
            h = jnp.maximum(h, 0.0)
            U1 = mm(S, h)
            U2 = 2.0 * mm(S, U1) - h
            o = (mm(h, W2_ref[0]) + mm(U1, W2_ref[1]) + mm(U2, W2_ref[2])
                 + b2v)
            c1 = lax.dot_general(o, Wc1, (((0,), (0,)), ((), ())),
                                 precision=hi,
                                 preferred_element_type=jnp.float32)
            c1 = jnp.maximum(c1 + bc1, 0.0)
            c2 = jnp.maximum(mm(c1, Wc2) + bc2, 0.0)
            return mm(c2, Wc3) + bc3

        A1 = parts_ref[0, 0] + parts_ref[1, 0]
        A2 = parts_ref[0, 1] + parts_ref[1, 1]
        pad_x = ((0, _NP - _N), (0, 0))
        out1_ref[...] = run(A1, jnp.pad(x1_ref[...], pad_x))
        out2_ref[...] = run(A2, jnp.pad(x2_ref[...], pad_x))

    return pl.pallas_call(
        body,
        out_shape=[jax.ShapeDtypeStruct((2, 1), jnp.float32)] * 2,
    )(parts, x1, x2, W1, b1, W2, b2, Wc1p, bc1, Wc2, bc2, Wc3, bc3)


def kernel(x1, edge_index1, edge_attr1, x2, edge_index2, edge_attr2, W1, b1,
           W2, b2, Wc1, bc1, Wc2, bc2, Wc3, bc3):
    def prep(ei, ew):
        srcp = jnp.pad(ei[0].astype(jnp.int32), (0, _EPAD - _E))
        dstp = jnp.pad(ei[1].astype(jnp.int32), (0, _EPAD - _E))
        ewp = jnp.pad(ew.astype(jnp.float32), (0, _EPAD - _E))
        return (srcp.reshape(_NT, _RPG, 128), dstp.reshape(_NT, _RPG, 128),
                ewp.reshape(_NT, _RPG, 128))

    s1, d1, e1 = prep(edge_index1, edge_attr1)
    s2, d2, e2 = prep(edge_index2, edge_attr2)
    src = jnp.concatenate([s1, s2], axis=1)
    dst = jnp.concatenate([d1, d2], axis=1)
    ewc = jnp.concatenate([e1, e2], axis=1)
    zeros = jnp.zeros((_SLC,), jnp.float32)
    parts = _sc_build_adj(src, dst, ewc, zeros).reshape(2, 2, _NP, _NP)
    out1, out2 = _tc_forward(parts, x1, x2, W1, b1, W2, b2, Wc1, bc1,
                             Wc2, bc2, Wc3, bc3)
    return (out1, out2)
